# Initial kernel scaffold; baseline (speedup 1.0000x reference)
import functools

import jax
import jax.numpy as jnp
from jax import lax
from jax.experimental import pallas as pl
from jax.experimental.pallas import tpu as pltpu
from jax.experimental.pallas import tpu_sc as plsc

N = 10000
E = 320000
D_IN = 128
H = 8
F1 = 16
C = 40
HF1 = H * F1
HC = H * C
NCORE = 2
NSUB = 16
TILE_ROWS = N // NSUB
K = 80
W1ROW = HF1 + 16
W2ROW = 160 + 16

_f32 = jnp.float32


def _prep1_body(x_ref, w1_ref, al_ref, ar_ref, rep_ref,
                h_ref, el_ref, er_ref, w_ref, winit_ref):
    x = x_ref[...]
    h = jnp.dot(x, w1_ref[...], preferred_element_type=_f32)
    el = jnp.dot(h, al_ref[...], preferred_element_type=_f32)
    er = jnp.dot(h, ar_ref[...], preferred_element_type=_f32)
    z = el + er
    w = jnp.exp(jnp.maximum(z, 0.2 * z))
    wex = jnp.dot(w, rep_ref[...], preferred_element_type=_f32)
    h_ref[...] = h
    el_ref[...] = el
    er_ref[...] = er
    w_ref[...] = w
    winit_ref[...] = h * wex


def _prep1(x, W1, Al, Ar, Rep, rb=2000):
    grid = (N // rb,)
    return pl.pallas_call(
        _prep1_body,
        grid=grid,
        in_specs=[
            pl.BlockSpec((rb, D_IN), lambda i: (i, 0)),
            pl.BlockSpec((D_IN, HF1), lambda i: (0, 0)),
            pl.BlockSpec((HF1, H), lambda i: (0, 0)),
            pl.BlockSpec((HF1, H), lambda i: (0, 0)),
            pl.BlockSpec((H, HF1), lambda i: (0, 0)),
        ],
        out_specs=[
            pl.BlockSpec((rb, HF1), lambda i: (i, 0)),
            pl.BlockSpec((rb, H), lambda i: (i, 0)),
            pl.BlockSpec((rb, H), lambda i: (i, 0)),
            pl.BlockSpec((rb, H), lambda i: (i, 0)),
            pl.BlockSpec((rb, HF1), lambda i: (i, 0)),
        ],
        out_shape=[
            jax.ShapeDtypeStruct((N, HF1), _f32),
            jax.ShapeDtypeStruct((N, H), _f32),
            jax.ShapeDtypeStruct((N, H), _f32),
            jax.ShapeDtypeStruct((N, H), _f32),
            jax.ShapeDtypeStruct((N, HF1), _f32),
        ],
    )(x, W1, Al, Ar, Rep)


def _prep2_body(p_ref, b1_ref, rep1_ref, w2_ref, al_ref, ar_ref, rep2_ref,
                h2_ref, el_ref, er_ref, w_ref, winit_ref):
    acc = p_ref[0] + p_ref[1]
    num = acc[:, :HF1]
    den = acc[:, HF1:HF1 + H]
    denex = jnp.dot(den, rep1_ref[...], preferred_element_type=_f32)
    x2 = jnp.maximum(num / denex + b1_ref[...], 0.0)
    h2 = jnp.dot(x2, w2_ref[...], preferred_element_type=_f32)
    el = jnp.dot(h2, al_ref[...], preferred_element_type=_f32)
    er = jnp.dot(h2, ar_ref[...], preferred_element_type=_f32)
    z = el + er
    w = jnp.exp(jnp.maximum(z, 0.2 * z))
    wex = jnp.dot(w, rep2_ref[...], preferred_element_type=_f32)
    h2_ref[...] = h2
    el_ref[...] = el
    er_ref[...] = er
    w_ref[...] = w
    winit_ref[...] = h2 * wex


def _prep2(p, b1, Rep1, W2, Al2, Ar2, Rep2, rb=2000):
    grid = (N // rb,)
    return pl.pallas_call(
        _prep2_body,
        grid=grid,
        in_specs=[
            pl.BlockSpec((2, rb, W1ROW), lambda i: (0, i, 0)),
            pl.BlockSpec((1, HF1), lambda i: (0, 0)),
            pl.BlockSpec((H, HF1), lambda i: (0, 0)),
            pl.BlockSpec((HF1, HC), lambda i: (0, 0)),
            pl.BlockSpec((HC, H), lambda i: (0, 0)),
            pl.BlockSpec((HC, H), lambda i: (0, 0)),
            pl.BlockSpec((H, HC), lambda i: (0, 0)),
        ],
        out_specs=[
            pl.BlockSpec((rb, HC), lambda i: (i, 0)),
            pl.BlockSpec((rb, H), lambda i: (i, 0)),
            pl.BlockSpec((rb, H), lambda i: (i, 0)),
            pl.BlockSpec((rb, H), lambda i: (i, 0)),
            pl.BlockSpec((rb, HC), lambda i: (i, 0)),
        ],
        out_shape=[
            jax.ShapeDtypeStruct((N, HC), _f32),
            jax.ShapeDtypeStruct((N, H), _f32),
            jax.ShapeDtypeStruct((N, H), _f32),
            jax.ShapeDtypeStruct((N, H), _f32),
            jax.ShapeDtypeStruct((N, HC), _f32),
        ],
    )(p, b1, Rep1, W2, Al2, Ar2, Rep2)


def _final_body(nume_ref, dens_ref, rep_ref, s_ref, b2m_ref, o_ref):
    rec = 1.0 / dens_ref[...]
    recex = jnp.dot(rec, rep_ref[...], preferred_element_type=_f32)
    contrib = nume_ref[...] * recex
    out = jnp.dot(contrib, s_ref[...], preferred_element_type=_f32)
    o_ref[...] = out * (1.0 / H) + b2m_ref[...]


def _final(nume, dens, Rep2, S, b2m, rb=2000):
    grid = (N // rb,)
    return pl.pallas_call(
        _final_body,
        grid=grid,
        in_specs=[
            pl.BlockSpec((rb, HC), lambda i: (i, 0)),
            pl.BlockSpec((rb, H), lambda i: (i, 0)),
            pl.BlockSpec((H, HC), lambda i: (0, 0)),
            pl.BlockSpec((HC, C), lambda i: (0, 0)),
            pl.BlockSpec((1, C), lambda i: (0, 0)),
        ],
        out_specs=pl.BlockSpec((rb, C), lambda i: (i, 0)),
        out_shape=jax.ShapeDtypeStruct((N, C), _f32),
    )(nume, dens, Rep2, S, b2m)


def _edge_kernel(row_w, nheads, per_core_edges, htab, ert, init, src, dst):
    mesh = plsc.VectorSubcoreMesh(core_axis_name="c", subcore_axis_name="s")
    per_tile = per_core_edges // NSUB
    nch = per_tile // K
    nfeat = row_w - 16
    nf = nfeat // nheads

    @functools.partial(
        pl.kernel,
        out_type=jax.ShapeDtypeStruct((NCORE, N, row_w), _f32),
        mesh=mesh,
        scratch_types=[
            pltpu.VMEM((K, row_w), _f32),
            pltpu.VMEM((K, 16), _f32),
            pltpu.VMEM((K,), jnp.int32),
            pltpu.VMEM((K,), jnp.int32),
            pltpu.VMEM_SHARED((N, row_w), _f32),
            pltpu.SemaphoreType.DMA,
            pltpu.SemaphoreType.DMA,
        ],
    )
    def k(htab_hbm, ert_hbm, init_hbm, src_hbm, dst_hbm, out_hbm,
          gbuf, ebuf, sbuf, dbuf, acc, sem1, sem2):
        c = lax.axis_index("c")
        s = lax.axis_index("s")
        r0 = s * TILE_ROWS
        pltpu.sync_copy(init_hbm.at[c].at[pl.ds(r0, TILE_ROWS)],
                        acc.at[pl.ds(r0, TILE_ROWS)])
        plsc.subcore_barrier()

        base = c * per_core_edges + s * per_tile

        @pl.loop(0, nch)
        def _chunk(i):
            e0 = base + i * K
            pltpu.sync_copy(src_hbm.at[pl.ds(e0, K)], sbuf)
            pltpu.sync_copy(dst_hbm.at[pl.ds(e0, K)], dbuf)
            cp1 = pltpu.async_copy(htab_hbm.at[sbuf], gbuf, sem1)
            cp2 = pltpu.async_copy(ert_hbm.at[dbuf], ebuf, sem2)
            cp1.wait()
            cp2.wait()

            @pl.loop(0, K)
            def _edge(j):
                el = gbuf[j, pl.ds(nfeat, 16)]
                er = ebuf[j, pl.ds(0, 16)]
                z = el + er
                w = jnp.exp(jnp.maximum(z, 0.2 * z))
                gbuf[j, pl.ds(nfeat, 16)] = w
                for hh in range(nheads):
                    ws = gbuf[j, nfeat + hh]
                    for q in range(nf // 16):
                        sl = pl.ds(hh * nf + q * 16, 16)
                        gbuf[j, sl] = gbuf[j, sl] * ws

            pltpu.sync_copy(gbuf, acc.at[dbuf], add=True)

        plsc.subcore_barrier()
        pltpu.sync_copy(acc.at[pl.ds(r0, TILE_ROWS)],
                        out_hbm.at[c].at[pl.ds(r0, TILE_ROWS)])

    return k(htab, ert, init, src, dst)


def _head_reduce_mat(a):
    heads, f = a.shape
    eye = jnp.eye(heads, dtype=_f32)
    return (a[:, :, None] * eye[:, None, :]).reshape(heads * f, heads)


def _repeat_mat(heads, f):
    eye = jnp.eye(heads, dtype=_f32)
    return jnp.repeat(eye, f, axis=1)


def _headsum_mat(heads, f):
    return jnp.tile(jnp.eye(f, dtype=_f32), (heads, 1))


def kernel(features, edge_index, W1, a_l1, a_r1, b1, W2, a_l2, a_r2, b2):
    src = edge_index[0].astype(jnp.int32)
    dst = edge_index[1].astype(jnp.int32)

    Al1 = _head_reduce_mat(a_l1)
    Ar1 = _head_reduce_mat(a_r1)
    Rep1 = _repeat_mat(H, F1)
    Al2 = _head_reduce_mat(a_l2)
    Ar2 = _head_reduce_mat(a_r2)
    Rep2 = _repeat_mat(H, C)
    S2 = _headsum_mat(H, C)
    b2m = jnp.mean(b2.reshape(H, C), axis=0, keepdims=True)

    h1, el1, er1, w1s, winit1 = _prep1(features, W1, Al1, Ar1, Rep1)
    zeros8 = jnp.zeros((N, 8), _f32)
    htab1 = jnp.concatenate([h1, el1, zeros8], axis=1)
    ert1 = jnp.concatenate([er1, zeros8], axis=1)
    init1 = jnp.stack([
        jnp.concatenate([winit1, w1s, jnp.ones((N, 8), _f32)], axis=1),
        jnp.zeros((N, W1ROW), _f32),
    ])
    p1 = _edge_kernel(W1ROW, H, E // 2, htab1, ert1, init1, src, dst)

    h2, el2, er2, w2s, winit2 = _prep2(p1, b1.reshape(1, HF1), Rep1,
                                       W2, Al2, Ar2, Rep2)
    halves_h, halves_e, halves_i = [], [], []
    zeros12 = jnp.zeros((N, 12), _f32)
    for c in range(NCORE):
        f0, f1b = c * 160, (c + 1) * 160
        h0, h1b = c * 4, (c + 1) * 4
        halves_h.append(jnp.concatenate(
            [h2[:, f0:f1b], el2[:, h0:h1b], zeros12], axis=1))
        halves_e.append(jnp.concatenate(
            [er2[:, h0:h1b], zeros12], axis=1))
        halves_i.append(jnp.concatenate(
            [winit2[:, f0:f1b], w2s[:, h0:h1b], jnp.ones((N, 12), _f32)],
            axis=1))
    htab2 = jnp.concatenate(halves_h, axis=0)
    ert2 = jnp.concatenate(halves_e, axis=0)
    init2 = jnp.stack(halves_i)
    src2 = jnp.concatenate([src, src + N])
    dst2 = jnp.concatenate([dst, dst + N])
    p2 = _edge_kernel(W2ROW, 4, E, htab2, ert2, init2, src2, dst2)

    nume = jnp.concatenate([p2[0, :, :160], p2[1, :, :160]], axis=1)
    dens = jnp.concatenate([p2[0, :, 160:164], p2[1, :, 160:164]], axis=1)
    return _final(nume, dens, Rep2, S2, b2m)


# speedup vs baseline: 39.8085x; 39.8085x over previous
import functools

import jax
import jax.numpy as jnp
from jax import lax
from jax.experimental import pallas as pl
from jax.experimental.pallas import tpu as pltpu
from jax.experimental.pallas import tpu_sc as plsc

N = 10000
E = 320000
D_IN = 128
H = 8
F1 = 16
C = 40
HF1 = H * F1
HC = H * C
NCORE = 2
NSUB = 16
NPAD = 10112
TILE_ROWS = NPAD // NSUB
K = 80
W1ROW = HF1 + 16
W2ROW = 160 + 16

_f32 = jnp.float32


def _prep1_body(x_ref, w1_ref, al_ref, ar_ref, rep_ref,
                h_ref, el_ref, er_ref, w_ref, winit_ref):
    x = x_ref[...]
    h = jnp.dot(x, w1_ref[...], preferred_element_type=_f32)
    el = jnp.dot(h, al_ref[...], preferred_element_type=_f32)
    er = jnp.dot(h, ar_ref[...], preferred_element_type=_f32)
    z = el + er
    w = jnp.exp(jnp.maximum(z, 0.2 * z))
    wex = jnp.dot(w, rep_ref[...], preferred_element_type=_f32)
    h_ref[...] = h
    el_ref[...] = el
    er_ref[...] = er
    w_ref[...] = w
    winit_ref[...] = h * wex


def _prep1(x, W1, Al, Ar, Rep, rb=2000):
    grid = (N // rb,)
    return pl.pallas_call(
        _prep1_body,
        grid=grid,
        in_specs=[
            pl.BlockSpec((rb, D_IN), lambda i: (i, 0)),
            pl.BlockSpec((D_IN, HF1), lambda i: (0, 0)),
            pl.BlockSpec((HF1, H), lambda i: (0, 0)),
            pl.BlockSpec((HF1, H), lambda i: (0, 0)),
            pl.BlockSpec((H, HF1), lambda i: (0, 0)),
        ],
        out_specs=[
            pl.BlockSpec((rb, HF1), lambda i: (i, 0)),
            pl.BlockSpec((rb, H), lambda i: (i, 0)),
            pl.BlockSpec((rb, H), lambda i: (i, 0)),
            pl.BlockSpec((rb, H), lambda i: (i, 0)),
            pl.BlockSpec((rb, HF1), lambda i: (i, 0)),
        ],
        out_shape=[
            jax.ShapeDtypeStruct((N, HF1), _f32),
            jax.ShapeDtypeStruct((N, H), _f32),
            jax.ShapeDtypeStruct((N, H), _f32),
            jax.ShapeDtypeStruct((N, H), _f32),
            jax.ShapeDtypeStruct((N, HF1), _f32),
        ],
    )(x, W1, Al, Ar, Rep)


def _prep2_body(p_ref, b1_ref, rep1_ref, w2_ref, al_ref, ar_ref, rep2_ref,
                h2_ref, el_ref, er_ref, w_ref, winit_ref):
    acc = p_ref[0] + p_ref[1]
    num = acc[:, :HF1]
    den = acc[:, HF1:HF1 + H]
    denex = jnp.dot(den, rep1_ref[...], preferred_element_type=_f32)
    x2 = jnp.maximum(num / denex + b1_ref[...], 0.0)
    h2 = jnp.dot(x2, w2_ref[...], preferred_element_type=_f32)
    el = jnp.dot(h2, al_ref[...], preferred_element_type=_f32)
    er = jnp.dot(h2, ar_ref[...], preferred_element_type=_f32)
    z = el + er
    w = jnp.exp(jnp.maximum(z, 0.2 * z))
    wex = jnp.dot(w, rep2_ref[...], preferred_element_type=_f32)
    h2_ref[...] = h2
    el_ref[...] = el
    er_ref[...] = er
    w_ref[...] = w
    winit_ref[...] = h2 * wex


def _prep2(p, b1, Rep1, W2, Al2, Ar2, Rep2, rb=2000):
    grid = (N // rb,)
    return pl.pallas_call(
        _prep2_body,
        grid=grid,
        in_specs=[
            pl.BlockSpec((2, rb, W1ROW), lambda i: (0, i, 0)),
            pl.BlockSpec((1, HF1), lambda i: (0, 0)),
            pl.BlockSpec((H, HF1), lambda i: (0, 0)),
            pl.BlockSpec((HF1, HC), lambda i: (0, 0)),
            pl.BlockSpec((HC, H), lambda i: (0, 0)),
            pl.BlockSpec((HC, H), lambda i: (0, 0)),
            pl.BlockSpec((H, HC), lambda i: (0, 0)),
        ],
        out_specs=[
            pl.BlockSpec((rb, HC), lambda i: (i, 0)),
            pl.BlockSpec((rb, H), lambda i: (i, 0)),
            pl.BlockSpec((rb, H), lambda i: (i, 0)),
            pl.BlockSpec((rb, H), lambda i: (i, 0)),
            pl.BlockSpec((rb, HC), lambda i: (i, 0)),
        ],
        out_shape=[
            jax.ShapeDtypeStruct((N, HC), _f32),
            jax.ShapeDtypeStruct((N, H), _f32),
            jax.ShapeDtypeStruct((N, H), _f32),
            jax.ShapeDtypeStruct((N, H), _f32),
            jax.ShapeDtypeStruct((N, HC), _f32),
        ],
    )(p, b1, Rep1, W2, Al2, Ar2, Rep2)


def _final_body(nume_ref, dens_ref, rep_ref, s_ref, b2m_ref, o_ref):
    rec = 1.0 / dens_ref[...]
    recex = jnp.dot(rec, rep_ref[...], preferred_element_type=_f32)
    contrib = nume_ref[...] * recex
    out = jnp.dot(contrib, s_ref[...], preferred_element_type=_f32)
    o_ref[...] = out * (1.0 / H) + b2m_ref[...]


def _final(nume, dens, Rep2, S, b2m, rb=2000):
    grid = (N // rb,)
    return pl.pallas_call(
        _final_body,
        grid=grid,
        in_specs=[
            pl.BlockSpec((rb, HC), lambda i: (i, 0)),
            pl.BlockSpec((rb, H), lambda i: (i, 0)),
            pl.BlockSpec((H, HC), lambda i: (0, 0)),
            pl.BlockSpec((HC, C), lambda i: (0, 0)),
            pl.BlockSpec((1, C), lambda i: (0, 0)),
        ],
        out_specs=pl.BlockSpec((rb, C), lambda i: (i, 0)),
        out_shape=jax.ShapeDtypeStruct((N, C), _f32),
    )(nume, dens, Rep2, S, b2m)


def _edge_kernel(row_w, nheads, per_core_edges, htab, ert, init,
                 src, dst_g, dst_s):
    mesh = plsc.VectorSubcoreMesh(core_axis_name="c", subcore_axis_name="s")
    per_tile = per_core_edges // NSUB
    nch = per_tile // K
    nfeat = row_w - 16
    nf = nfeat // nheads

    @functools.partial(
        pl.kernel,
        out_type=jax.ShapeDtypeStruct((NCORE, NPAD, row_w), _f32),
        mesh=mesh,
        scratch_types=[
            pltpu.VMEM((K, row_w), _f32),
            pltpu.VMEM((K, 16), _f32),
            pltpu.VMEM((K,), jnp.int32),
            pltpu.VMEM((K,), jnp.int32),
            pltpu.VMEM((K,), jnp.int32),
            pltpu.VMEM_SHARED((NPAD, row_w), _f32),
            pltpu.SemaphoreType.DMA,
            pltpu.SemaphoreType.DMA,
        ],
        compiler_params=pltpu.CompilerParams(use_tc_tiling_on_sc=False),
    )
    def k(htab_hbm, ert_hbm, init_hbm, src_hbm, dstg_hbm, dsts_hbm, out_hbm,
          gbuf, ebuf, sbuf, dbuf, dsbuf, acc, sem1, sem2):
        c = lax.axis_index("c")
        s = lax.axis_index("s")
        r0 = s * TILE_ROWS
        pltpu.sync_copy(init_hbm.at[c].at[pl.ds(r0, TILE_ROWS)],
                        acc.at[pl.ds(r0, TILE_ROWS)])
        plsc.subcore_barrier()

        base = c * per_core_edges + s * per_tile

        @pl.loop(0, nch)
        def _chunk(i):
            e0 = base + i * K
            pltpu.sync_copy(src_hbm.at[pl.ds(e0, K)], sbuf)
            pltpu.sync_copy(dstg_hbm.at[pl.ds(e0, K)], dbuf)
            pltpu.sync_copy(dsts_hbm.at[pl.ds(e0, K)], dsbuf)
            cp1 = pltpu.async_copy(htab_hbm.at[sbuf], gbuf, sem1)
            cp2 = pltpu.async_copy(ert_hbm.at[dbuf], ebuf, sem2)
            cp1.wait()
            cp2.wait()

            @pl.loop(0, K)
            def _edge(j):
                el = gbuf[j, pl.ds(nfeat, 16)]
                er = ebuf[j, pl.ds(0, 16)]
                z = el + er
                w = jnp.exp(jnp.maximum(z, 0.2 * z))
                gbuf[j, pl.ds(nfeat, 16)] = w
                for t in range(nfeat // 16):
                    lo = (16 * t) // nf
                    hi = (16 * t + 15) // nf
                    sl = pl.ds(16 * t, 16)
                    if lo == hi:
                        gbuf[j, sl] = gbuf[j, sl] * w[lo]
                    else:
                        lanes = lax.iota(jnp.int32, 16)
                        wv = jnp.where(lanes < (nf * hi - 16 * t),
                                       w[lo], w[hi])
                        gbuf[j, sl] = gbuf[j, sl] * wv

            pltpu.sync_copy(gbuf, acc.at[dsbuf], add=True)

        plsc.subcore_barrier()
        pltpu.sync_copy(acc.at[pl.ds(r0, TILE_ROWS)],
                        out_hbm.at[c].at[pl.ds(r0, TILE_ROWS)])

    return k(htab, ert, init, src, dst_g, dst_s)


def _head_reduce_mat(a):
    heads, f = a.shape
    eye = jnp.eye(heads, dtype=_f32)
    return (a[:, :, None] * eye[:, None, :]).reshape(heads * f, heads)


def _repeat_mat(heads, f):
    eye = jnp.eye(heads, dtype=_f32)
    return jnp.repeat(eye, f, axis=1)


def _headsum_mat(heads, f):
    return jnp.tile(jnp.eye(f, dtype=_f32), (heads, 1))


def kernel(features, edge_index, W1, a_l1, a_r1, b1, W2, a_l2, a_r2, b2):
    src = edge_index[0].astype(jnp.int32)
    dst = edge_index[1].astype(jnp.int32)

    Al1 = _head_reduce_mat(a_l1)
    Ar1 = _head_reduce_mat(a_r1)
    Rep1 = _repeat_mat(H, F1)
    Al2 = _head_reduce_mat(a_l2)
    Ar2 = _head_reduce_mat(a_r2)
    Rep2 = _repeat_mat(H, C)
    S2 = _headsum_mat(H, C)
    b2m = jnp.mean(b2.reshape(H, C), axis=0, keepdims=True)

    h1, el1, er1, w1s, winit1 = _prep1(features, W1, Al1, Ar1, Rep1)
    zeros8 = jnp.zeros((N, 8), _f32)
    htab1 = jnp.concatenate([h1, el1, zeros8], axis=1)
    ert1 = jnp.concatenate([er1, zeros8], axis=1)
    init1 = jnp.stack([
        jnp.concatenate([winit1, w1s, jnp.ones((N, 8), _f32)], axis=1),
        jnp.zeros((N, W1ROW), _f32),
    ])
    init1 = jnp.pad(init1, ((0, 0), (0, NPAD - N), (0, 0)))
    p1 = _edge_kernel(W1ROW, H, E // 2, htab1, ert1, init1, src, dst, dst)
    p1 = p1[:, :N]

    h2, el2, er2, w2s, winit2 = _prep2(p1, b1.reshape(1, HF1), Rep1,
                                       W2, Al2, Ar2, Rep2)
    halves_h, halves_e, halves_i = [], [], []
    zeros12 = jnp.zeros((N, 12), _f32)
    for c in range(NCORE):
        f0, f1b = c * 160, (c + 1) * 160
        h0, h1b = c * 4, (c + 1) * 4
        halves_h.append(jnp.concatenate(
            [h2[:, f0:f1b], el2[:, h0:h1b], zeros12], axis=1))
        halves_e.append(jnp.concatenate(
            [er2[:, h0:h1b], zeros12], axis=1))
        halves_i.append(jnp.concatenate(
            [winit2[:, f0:f1b], w2s[:, h0:h1b], jnp.ones((N, 12), _f32)],
            axis=1))
    htab2 = jnp.concatenate(halves_h, axis=0)
    ert2 = jnp.concatenate(halves_e, axis=0)
    init2 = jnp.pad(jnp.stack(halves_i), ((0, 0), (0, NPAD - N), (0, 0)))
    src2 = jnp.concatenate([src, src + N])
    dst2 = jnp.concatenate([dst, dst + N])
    dst2s = jnp.concatenate([dst, dst])
    p2 = _edge_kernel(W2ROW, 4, E, htab2, ert2, init2, src2, dst2, dst2s)

    nume = jnp.concatenate([p2[0, :N, :160], p2[1, :N, :160]], axis=1)
    dens = jnp.concatenate([p2[0, :N, 160:164], p2[1, :N, 160:164]], axis=1)
    return _final(nume, dens, Rep2, S2, b2m)


# speedup vs baseline: 50.7502x; 1.2749x over previous
import functools

import jax
import jax.numpy as jnp
from jax import lax
from jax.experimental import pallas as pl
from jax.experimental.pallas import tpu as pltpu
from jax.experimental.pallas import tpu_sc as plsc

N = 10000
E = 320000
D_IN = 128
H = 8
F1 = 16
C = 40
HF1 = H * F1
HC = H * C
NCORE = 2
NSUB = 16
NPAD = 10112
TILE_ROWS = NPAD // NSUB
K = 40
W1ROW = HF1 + 16
W2ROW = 160 + 16

_f32 = jnp.float32


def _prep1_body(x_ref, w1_ref, al_ref, ar_ref, rep_ref,
                h_ref, el_ref, er_ref, w_ref, winit_ref):
    x = x_ref[...]
    h = jnp.dot(x, w1_ref[...], preferred_element_type=_f32)
    el = jnp.dot(h, al_ref[...], preferred_element_type=_f32)
    er = jnp.dot(h, ar_ref[...], preferred_element_type=_f32)
    z = el + er
    w = jnp.exp(jnp.maximum(z, 0.2 * z))
    wex = jnp.dot(w, rep_ref[...], preferred_element_type=_f32)
    h_ref[...] = h
    el_ref[...] = el
    er_ref[...] = er
    w_ref[...] = w
    winit_ref[...] = h * wex


def _prep1(x, W1, Al, Ar, Rep, rb=2000):
    grid = (N // rb,)
    return pl.pallas_call(
        _prep1_body,
        grid=grid,
        in_specs=[
            pl.BlockSpec((rb, D_IN), lambda i: (i, 0)),
            pl.BlockSpec((D_IN, HF1), lambda i: (0, 0)),
            pl.BlockSpec((HF1, H), lambda i: (0, 0)),
            pl.BlockSpec((HF1, H), lambda i: (0, 0)),
            pl.BlockSpec((H, HF1), lambda i: (0, 0)),
        ],
        out_specs=[
            pl.BlockSpec((rb, HF1), lambda i: (i, 0)),
            pl.BlockSpec((rb, H), lambda i: (i, 0)),
            pl.BlockSpec((rb, H), lambda i: (i, 0)),
            pl.BlockSpec((rb, H), lambda i: (i, 0)),
            pl.BlockSpec((rb, HF1), lambda i: (i, 0)),
        ],
        out_shape=[
            jax.ShapeDtypeStruct((N, HF1), _f32),
            jax.ShapeDtypeStruct((N, H), _f32),
            jax.ShapeDtypeStruct((N, H), _f32),
            jax.ShapeDtypeStruct((N, H), _f32),
            jax.ShapeDtypeStruct((N, HF1), _f32),
        ],
    )(x, W1, Al, Ar, Rep)


def _prep2_body(p_ref, b1_ref, rep1_ref, w2_ref, al_ref, ar_ref, rep2_ref,
                h2_ref, el_ref, er_ref, w_ref, winit_ref):
    acc = p_ref[0] + p_ref[1]
    num = acc[:, :HF1]
    den = acc[:, HF1:HF1 + H]
    denex = jnp.dot(den, rep1_ref[...], preferred_element_type=_f32)
    x2 = jnp.maximum(num / denex + b1_ref[...], 0.0)
    h2 = jnp.dot(x2, w2_ref[...], preferred_element_type=_f32)
    el = jnp.dot(h2, al_ref[...], preferred_element_type=_f32)
    er = jnp.dot(h2, ar_ref[...], preferred_element_type=_f32)
    z = el + er
    w = jnp.exp(jnp.maximum(z, 0.2 * z))
    wex = jnp.dot(w, rep2_ref[...], preferred_element_type=_f32)
    h2_ref[...] = h2
    el_ref[...] = el
    er_ref[...] = er
    w_ref[...] = w
    winit_ref[...] = h2 * wex


def _prep2(p, b1, Rep1, W2, Al2, Ar2, Rep2, rb=2000):
    grid = (N // rb,)
    return pl.pallas_call(
        _prep2_body,
        grid=grid,
        in_specs=[
            pl.BlockSpec((2, rb, W1ROW), lambda i: (0, i, 0)),
            pl.BlockSpec((1, HF1), lambda i: (0, 0)),
            pl.BlockSpec((H, HF1), lambda i: (0, 0)),
            pl.BlockSpec((HF1, HC), lambda i: (0, 0)),
            pl.BlockSpec((HC, H), lambda i: (0, 0)),
            pl.BlockSpec((HC, H), lambda i: (0, 0)),
            pl.BlockSpec((H, HC), lambda i: (0, 0)),
        ],
        out_specs=[
            pl.BlockSpec((rb, HC), lambda i: (i, 0)),
            pl.BlockSpec((rb, H), lambda i: (i, 0)),
            pl.BlockSpec((rb, H), lambda i: (i, 0)),
            pl.BlockSpec((rb, H), lambda i: (i, 0)),
            pl.BlockSpec((rb, HC), lambda i: (i, 0)),
        ],
        out_shape=[
            jax.ShapeDtypeStruct((N, HC), _f32),
            jax.ShapeDtypeStruct((N, H), _f32),
            jax.ShapeDtypeStruct((N, H), _f32),
            jax.ShapeDtypeStruct((N, H), _f32),
            jax.ShapeDtypeStruct((N, HC), _f32),
        ],
    )(p, b1, Rep1, W2, Al2, Ar2, Rep2)


def _final_body(nume_ref, dens_ref, rep_ref, s_ref, b2m_ref, o_ref):
    rec = 1.0 / dens_ref[...]
    recex = jnp.dot(rec, rep_ref[...], preferred_element_type=_f32)
    contrib = nume_ref[...] * recex
    out = jnp.dot(contrib, s_ref[...], preferred_element_type=_f32)
    o_ref[...] = out * (1.0 / H) + b2m_ref[...]


def _final(nume, dens, Rep2, S, b2m, rb=2000):
    grid = (N // rb,)
    return pl.pallas_call(
        _final_body,
        grid=grid,
        in_specs=[
            pl.BlockSpec((rb, HC), lambda i: (i, 0)),
            pl.BlockSpec((rb, H), lambda i: (i, 0)),
            pl.BlockSpec((H, HC), lambda i: (0, 0)),
            pl.BlockSpec((HC, C), lambda i: (0, 0)),
            pl.BlockSpec((1, C), lambda i: (0, 0)),
        ],
        out_specs=pl.BlockSpec((rb, C), lambda i: (i, 0)),
        out_shape=jax.ShapeDtypeStruct((N, C), _f32),
    )(nume, dens, Rep2, S, b2m)


def _edge_kernel(row_w, nheads, per_core_edges, htab, ert, init,
                 src, dst_g, dst_s):
    mesh = plsc.VectorSubcoreMesh(core_axis_name="c", subcore_axis_name="s")
    per_tile = per_core_edges // NSUB
    nch = per_tile // K
    nfeat = row_w - 16
    nf = nfeat // nheads
    npair = nch // 2
    assert nch % 2 == 0

    @functools.partial(
        pl.kernel,
        out_type=jax.ShapeDtypeStruct((NCORE, NPAD, row_w), _f32),
        mesh=mesh,
        scratch_types=[
            pltpu.VMEM((K, row_w), _f32),
            pltpu.VMEM((K, row_w), _f32),
            pltpu.VMEM((K, 16), _f32),
            pltpu.VMEM((K, 16), _f32),
            pltpu.VMEM((3, K), jnp.int32),
            pltpu.VMEM((3, K), jnp.int32),
            pltpu.VMEM_SHARED((NPAD, row_w), _f32),
            pltpu.SemaphoreType.DMA,
            pltpu.SemaphoreType.DMA,
            pltpu.SemaphoreType.DMA,
            pltpu.SemaphoreType.DMA,
            pltpu.SemaphoreType.DMA,
            pltpu.SemaphoreType.DMA,
        ],
        compiler_params=pltpu.CompilerParams(use_tc_tiling_on_sc=False),
    )
    def k(htab_hbm, ert_hbm, init_hbm, idx_hbm, out_hbm,
          gbufa, gbufb, ebufa, ebufb, ibufa, ibufb, acc,
          semah, semae, sembh, sembe, isema, isemb):
        c = lax.axis_index("c")
        s = lax.axis_index("s")
        r0 = s * TILE_ROWS
        pltpu.sync_copy(init_hbm.at[c].at[pl.ds(r0, TILE_ROWS)],
                        acc.at[pl.ds(r0, TILE_ROWS)])
        plsc.subcore_barrier()

        myidx = idx_hbm.at[c].at[s]

        def issueg(ib, gb, eb, semh, seme):
            pltpu.async_copy(htab_hbm.at[ib.at[0]], gb, semh)
            pltpu.async_copy(ert_hbm.at[ib.at[1]], eb, seme)

        def waitg(gb, eb, semh, seme):
            pltpu.make_async_copy(htab_hbm.at[pl.ds(0, K)], gb, semh).wait()
            pltpu.make_async_copy(ert_hbm.at[pl.ds(0, K)], eb, seme).wait()

        def ifetch(ci, ib, isem):
            pltpu.async_copy(myidx.at[ci], ib, isem)

        def iwait(ib, isem):
            pltpu.make_async_copy(myidx.at[0], ib, isem).wait()

        def do_chunk(ib, gb, eb):
            @pl.loop(0, K, step=4)
            def _edge(j0):
                for u in range(4):
                    j = j0 + u
                    el = gb[j, pl.ds(nfeat, 16)]
                    er = eb[j, pl.ds(0, 16)]
                    z = el + er
                    w = jnp.exp(jnp.maximum(z, 0.2 * z))
                    gb[j, pl.ds(nfeat, 16)] = w
                    for t in range(nfeat // 16):
                        lo = (16 * t) // nf
                        hi = (16 * t + 15) // nf
                        sl = pl.ds(16 * t, 16)
                        if lo == hi:
                            gb[j, sl] = gb[j, sl] * w[lo]
                        else:
                            lanes = lax.iota(jnp.int32, 16)
                            wv = jnp.where(lanes < (nf * hi - 16 * t),
                                           w[lo], w[hi])
                            gb[j, sl] = gb[j, sl] * wv

            pltpu.sync_copy(gb, acc.at[ib.at[2]], add=True)

        pltpu.sync_copy(myidx.at[0], ibufa)
        issueg(ibufa, gbufa, ebufa, semah, semae)
        ifetch(1, ibufb, isemb)

        @pl.loop(0, npair)
        def _pair(i):
            c0 = 2 * i
            iwait(ibufb, isemb)
            issueg(ibufb, gbufb, ebufb, sembh, sembe)
            waitg(gbufa, ebufa, semah, semae)
            do_chunk(ibufa, gbufa, ebufa)

            @pl.when(c0 + 2 < nch)
            def _():
                ifetch(c0 + 2, ibufa, isema)

            waitg(gbufb, ebufb, sembh, sembe)
            do_chunk(ibufb, gbufb, ebufb)

            @pl.when(c0 + 2 < nch)
            def _():
                iwait(ibufa, isema)
                issueg(ibufa, gbufa, ebufa, semah, semae)

            @pl.when(c0 + 3 < nch)
            def _():
                ifetch(c0 + 3, ibufb, isemb)

        plsc.subcore_barrier()
        pltpu.sync_copy(acc.at[pl.ds(r0, TILE_ROWS)],
                        out_hbm.at[c].at[pl.ds(r0, TILE_ROWS)])

    idx = jnp.stack([src.reshape(NCORE, NSUB, nch, K),
                     dst_g.reshape(NCORE, NSUB, nch, K),
                     dst_s.reshape(NCORE, NSUB, nch, K)], axis=3)
    return k(htab, ert, init, idx)


def _head_reduce_mat(a):
    heads, f = a.shape
    eye = jnp.eye(heads, dtype=_f32)
    return (a[:, :, None] * eye[:, None, :]).reshape(heads * f, heads)


def _repeat_mat(heads, f):
    eye = jnp.eye(heads, dtype=_f32)
    return jnp.repeat(eye, f, axis=1)


def _headsum_mat(heads, f):
    return jnp.tile(jnp.eye(f, dtype=_f32), (heads, 1))


def kernel(features, edge_index, W1, a_l1, a_r1, b1, W2, a_l2, a_r2, b2):
    src = edge_index[0].astype(jnp.int32)
    dst = edge_index[1].astype(jnp.int32)

    Al1 = _head_reduce_mat(a_l1)
    Ar1 = _head_reduce_mat(a_r1)
    Rep1 = _repeat_mat(H, F1)
    Al2 = _head_reduce_mat(a_l2)
    Ar2 = _head_reduce_mat(a_r2)
    Rep2 = _repeat_mat(H, C)
    S2 = _headsum_mat(H, C)
    b2m = jnp.mean(b2.reshape(H, C), axis=0, keepdims=True)

    h1, el1, er1, w1s, winit1 = _prep1(features, W1, Al1, Ar1, Rep1)
    zeros8 = jnp.zeros((N, 8), _f32)
    htab1 = jnp.concatenate([h1, el1, zeros8], axis=1)
    ert1 = jnp.concatenate([er1, zeros8], axis=1)
    init1 = jnp.stack([
        jnp.concatenate([winit1, w1s, jnp.ones((N, 8), _f32)], axis=1),
        jnp.zeros((N, W1ROW), _f32),
    ])
    init1 = jnp.pad(init1, ((0, 0), (0, NPAD - N), (0, 0)))
    p1 = _edge_kernel(W1ROW, H, E // 2, htab1, ert1, init1, src, dst, dst)
    p1 = p1[:, :N]

    h2, el2, er2, w2s, winit2 = _prep2(p1, b1.reshape(1, HF1), Rep1,
                                       W2, Al2, Ar2, Rep2)
    halves_h, halves_e, halves_i = [], [], []
    zeros12 = jnp.zeros((N, 12), _f32)
    for c in range(NCORE):
        f0, f1b = c * 160, (c + 1) * 160
        h0, h1b = c * 4, (c + 1) * 4
        halves_h.append(jnp.concatenate(
            [h2[:, f0:f1b], el2[:, h0:h1b], zeros12], axis=1))
        halves_e.append(jnp.concatenate(
            [er2[:, h0:h1b], zeros12], axis=1))
        halves_i.append(jnp.concatenate(
            [winit2[:, f0:f1b], w2s[:, h0:h1b], jnp.ones((N, 12), _f32)],
            axis=1))
    htab2 = jnp.concatenate(halves_h, axis=0)
    ert2 = jnp.concatenate(halves_e, axis=0)
    init2 = jnp.pad(jnp.stack(halves_i), ((0, 0), (0, NPAD - N), (0, 0)))
    src2 = jnp.concatenate([src, src + N])
    dst2 = jnp.concatenate([dst, dst + N])
    dst2s = jnp.concatenate([dst, dst])
    p2 = _edge_kernel(W2ROW, 4, E, htab2, ert2, init2, src2, dst2, dst2s)

    nume = jnp.concatenate([p2[0, :N, :160], p2[1, :N, :160]], axis=1)
    dens = jnp.concatenate([p2[0, :N, 160:164], p2[1, :N, 160:164]], axis=1)
    return _final(nume, dens, Rep2, S2, b2m)


# speedup vs baseline: 53.5032x; 1.0542x over previous
import functools

import jax
import jax.numpy as jnp
from jax import lax
from jax.experimental import pallas as pl
from jax.experimental.pallas import tpu as pltpu
from jax.experimental.pallas import tpu_sc as plsc

N = 10000
E = 320000
D_IN = 128
H = 8
F1 = 16
C = 40
HF1 = H * F1
HC = H * C
NCORE = 2
NSUB = 16
NPAD = 10112
TILE_ROWS = NPAD // NSUB
K = 40
W1ROW = HF1 + 16
W2ROW = 160 + 16

_f32 = jnp.float32


def _prep1_body(x_ref, w1_ref, al_ref, ar_ref, rep_ref,
                h_ref, el_ref, er_ref, w_ref, winit_ref):
    x = x_ref[...]
    h = jnp.dot(x, w1_ref[...], preferred_element_type=_f32)
    el = jnp.dot(h, al_ref[...], preferred_element_type=_f32)
    er = jnp.dot(h, ar_ref[...], preferred_element_type=_f32)
    z = el + er
    w = jnp.exp(jnp.maximum(z, 0.2 * z))
    wex = jnp.dot(w, rep_ref[...], preferred_element_type=_f32)
    h_ref[...] = h
    el_ref[...] = el
    er_ref[...] = er
    w_ref[...] = w
    winit_ref[...] = h * wex


def _prep1(x, W1, Al, Ar, Rep, rb=2000):
    grid = (N // rb,)
    return pl.pallas_call(
        _prep1_body,
        grid=grid,
        in_specs=[
            pl.BlockSpec((rb, D_IN), lambda i: (i, 0)),
            pl.BlockSpec((D_IN, HF1), lambda i: (0, 0)),
            pl.BlockSpec((HF1, H), lambda i: (0, 0)),
            pl.BlockSpec((HF1, H), lambda i: (0, 0)),
            pl.BlockSpec((H, HF1), lambda i: (0, 0)),
        ],
        out_specs=[
            pl.BlockSpec((rb, HF1), lambda i: (i, 0)),
            pl.BlockSpec((rb, H), lambda i: (i, 0)),
            pl.BlockSpec((rb, H), lambda i: (i, 0)),
            pl.BlockSpec((rb, H), lambda i: (i, 0)),
            pl.BlockSpec((rb, HF1), lambda i: (i, 0)),
        ],
        out_shape=[
            jax.ShapeDtypeStruct((N, HF1), _f32),
            jax.ShapeDtypeStruct((N, H), _f32),
            jax.ShapeDtypeStruct((N, H), _f32),
            jax.ShapeDtypeStruct((N, H), _f32),
            jax.ShapeDtypeStruct((N, HF1), _f32),
        ],
    )(x, W1, Al, Ar, Rep)


def _prep2_body(p_ref, b1_ref, rep1_ref, w2_ref, al_ref, ar_ref, rep2_ref,
                h2_ref, el_ref, er_ref, w_ref, winit_ref):
    acc = p_ref[0] + p_ref[1]
    num = acc[:, :HF1]
    den = acc[:, HF1:HF1 + H]
    denex = jnp.dot(den, rep1_ref[...], preferred_element_type=_f32)
    x2 = jnp.maximum(num / denex + b1_ref[...], 0.0)
    h2 = jnp.dot(x2, w2_ref[...], preferred_element_type=_f32)
    el = jnp.dot(h2, al_ref[...], preferred_element_type=_f32)
    er = jnp.dot(h2, ar_ref[...], preferred_element_type=_f32)
    z = el + er
    w = jnp.exp(jnp.maximum(z, 0.2 * z))
    wex = jnp.dot(w, rep2_ref[...], preferred_element_type=_f32)
    h2_ref[...] = h2
    el_ref[...] = el
    er_ref[...] = er
    w_ref[...] = w
    winit_ref[...] = h2 * wex


def _prep2(p, b1, Rep1, W2, Al2, Ar2, Rep2, rb=2000):
    grid = (N // rb,)
    return pl.pallas_call(
        _prep2_body,
        grid=grid,
        in_specs=[
            pl.BlockSpec((2, rb, W1ROW), lambda i: (0, i, 0)),
            pl.BlockSpec((1, HF1), lambda i: (0, 0)),
            pl.BlockSpec((H, HF1), lambda i: (0, 0)),
            pl.BlockSpec((HF1, HC), lambda i: (0, 0)),
            pl.BlockSpec((HC, H), lambda i: (0, 0)),
            pl.BlockSpec((HC, H), lambda i: (0, 0)),
            pl.BlockSpec((H, HC), lambda i: (0, 0)),
        ],
        out_specs=[
            pl.BlockSpec((rb, HC), lambda i: (i, 0)),
            pl.BlockSpec((rb, H), lambda i: (i, 0)),
            pl.BlockSpec((rb, H), lambda i: (i, 0)),
            pl.BlockSpec((rb, H), lambda i: (i, 0)),
            pl.BlockSpec((rb, HC), lambda i: (i, 0)),
        ],
        out_shape=[
            jax.ShapeDtypeStruct((N, HC), _f32),
            jax.ShapeDtypeStruct((N, H), _f32),
            jax.ShapeDtypeStruct((N, H), _f32),
            jax.ShapeDtypeStruct((N, H), _f32),
            jax.ShapeDtypeStruct((N, HC), _f32),
        ],
    )(p, b1, Rep1, W2, Al2, Ar2, Rep2)


def _final_body(nume_ref, dens_ref, rep_ref, s_ref, b2m_ref, o_ref):
    rec = 1.0 / dens_ref[...]
    recex = jnp.dot(rec, rep_ref[...], preferred_element_type=_f32)
    contrib = nume_ref[...] * recex
    out = jnp.dot(contrib, s_ref[...], preferred_element_type=_f32)
    o_ref[...] = out * (1.0 / H) + b2m_ref[...]


def _final(nume, dens, Rep2, S, b2m, rb=2000):
    grid = (N // rb,)
    return pl.pallas_call(
        _final_body,
        grid=grid,
        in_specs=[
            pl.BlockSpec((rb, HC), lambda i: (i, 0)),
            pl.BlockSpec((rb, H), lambda i: (i, 0)),
            pl.BlockSpec((H, HC), lambda i: (0, 0)),
            pl.BlockSpec((HC, C), lambda i: (0, 0)),
            pl.BlockSpec((1, C), lambda i: (0, 0)),
        ],
        out_specs=pl.BlockSpec((rb, C), lambda i: (i, 0)),
        out_shape=jax.ShapeDtypeStruct((N, C), _f32),
    )(nume, dens, Rep2, S, b2m)


def _edge_kernel(row_w, nheads, per_core_edges, htab, ert, init,
                 src, dst_g, dst_s):
    mesh = plsc.VectorSubcoreMesh(core_axis_name="c", subcore_axis_name="s")
    per_tile = per_core_edges // NSUB
    nch = per_tile // K
    nfeat = row_w - 16
    nf = nfeat // nheads
    npair = nch // 2
    assert nch % 2 == 0

    @functools.partial(
        pl.kernel,
        out_type=jax.ShapeDtypeStruct((NCORE, NPAD, row_w), _f32),
        mesh=mesh,
        scratch_types=[
            pltpu.VMEM((K, row_w), _f32),
            pltpu.VMEM((K, row_w), _f32),
            pltpu.VMEM((K, 16), _f32),
            pltpu.VMEM((K, 16), _f32),
            pltpu.VMEM((3, K), jnp.int32),
            pltpu.VMEM((3, K), jnp.int32),
            pltpu.VMEM_SHARED((NPAD, row_w), _f32),
            pltpu.SemaphoreType.DMA,
            pltpu.SemaphoreType.DMA,
            pltpu.SemaphoreType.DMA,
            pltpu.SemaphoreType.DMA,
            pltpu.SemaphoreType.DMA,
            pltpu.SemaphoreType.DMA,
            pltpu.SemaphoreType.DMA,
            pltpu.SemaphoreType.DMA,
        ],
        compiler_params=pltpu.CompilerParams(use_tc_tiling_on_sc=False),
    )
    def k(htab_hbm, ert_hbm, init_hbm, idx_hbm, out_hbm,
          gbufa, gbufb, ebufa, ebufb, ibufa, ibufb, acc,
          semah, semae, sembh, sembe, isema, isemb, semas, sembs):
        c = lax.axis_index("c")
        s = lax.axis_index("s")
        r0 = s * TILE_ROWS
        pltpu.sync_copy(init_hbm.at[c].at[pl.ds(r0, TILE_ROWS)],
                        acc.at[pl.ds(r0, TILE_ROWS)])
        plsc.subcore_barrier()

        myidx = idx_hbm.at[c].at[s]

        def issueg(ib, gb, eb, semh, seme):
            pltpu.async_copy(htab_hbm.at[ib.at[0]], gb, semh)
            pltpu.async_copy(ert_hbm.at[ib.at[1]], eb, seme)

        def waitg(gb, eb, semh, seme):
            pltpu.make_async_copy(htab_hbm.at[pl.ds(0, K)], gb, semh).wait()
            pltpu.make_async_copy(ert_hbm.at[pl.ds(0, K)], eb, seme).wait()

        def ifetch(ci, ib, isem):
            pltpu.async_copy(myidx.at[ci], ib, isem)

        def iwait(ib, isem):
            pltpu.make_async_copy(myidx.at[0], ib, isem).wait()

        def do_chunk(ib, gb, eb, sems):
            @pl.loop(0, K, step=8)
            def _edge(j0):
                for u in range(8):
                    j = j0 + u
                    el = gb[j, pl.ds(nfeat, 16)]
                    er = eb[j, pl.ds(0, 16)]
                    z = el + er
                    w = jnp.exp(jnp.maximum(z, 0.2 * z))
                    gb[j, pl.ds(nfeat, 16)] = w
                    for t in range(nfeat // 16):
                        lo = (16 * t) // nf
                        hi = (16 * t + 15) // nf
                        sl = pl.ds(16 * t, 16)
                        if lo == hi:
                            gb[j, sl] = gb[j, sl] * w[lo]
                        else:
                            lanes = lax.iota(jnp.int32, 16)
                            wv = jnp.where(lanes < (nf * hi - 16 * t),
                                           w[lo], w[hi])
                            gb[j, sl] = gb[j, sl] * wv

            pltpu.async_copy(gb, acc.at[ib.at[2]], sems, add=True)

        def waits(gb, sems):
            pltpu.make_async_copy(gb, acc.at[pl.ds(0, K)], sems).wait()

        pltpu.sync_copy(myidx.at[0], ibufa)
        issueg(ibufa, gbufa, ebufa, semah, semae)
        ifetch(1, ibufb, isemb)

        @pl.loop(0, npair)
        def _pair(i):
            c0 = 2 * i

            @pl.when(i > 0)
            def _():
                waits(gbufb, sembs)

            iwait(ibufb, isemb)
            issueg(ibufb, gbufb, ebufb, sembh, sembe)
            waitg(gbufa, ebufa, semah, semae)
            do_chunk(ibufa, gbufa, ebufa, semas)

            @pl.when(c0 + 2 < nch)
            def _():
                ifetch(c0 + 2, ibufa, isema)

            waitg(gbufb, ebufb, sembh, sembe)
            do_chunk(ibufb, gbufb, ebufb, sembs)

            @pl.when(c0 + 2 < nch)
            def _():
                iwait(ibufa, isema)
                waits(gbufa, semas)
                issueg(ibufa, gbufa, ebufa, semah, semae)

            @pl.when(c0 + 3 < nch)
            def _():
                ifetch(c0 + 3, ibufb, isemb)

        waits(gbufa, semas)
        waits(gbufb, sembs)
        plsc.subcore_barrier()
        pltpu.sync_copy(acc.at[pl.ds(r0, TILE_ROWS)],
                        out_hbm.at[c].at[pl.ds(r0, TILE_ROWS)])

    idx = jnp.stack([src.reshape(NCORE, NSUB, nch, K),
                     dst_g.reshape(NCORE, NSUB, nch, K),
                     dst_s.reshape(NCORE, NSUB, nch, K)], axis=3)
    return k(htab, ert, init, idx)


def _head_reduce_mat(a):
    heads, f = a.shape
    eye = jnp.eye(heads, dtype=_f32)
    return (a[:, :, None] * eye[:, None, :]).reshape(heads * f, heads)


def _repeat_mat(heads, f):
    eye = jnp.eye(heads, dtype=_f32)
    return jnp.repeat(eye, f, axis=1)


def _headsum_mat(heads, f):
    return jnp.tile(jnp.eye(f, dtype=_f32), (heads, 1))


def kernel(features, edge_index, W1, a_l1, a_r1, b1, W2, a_l2, a_r2, b2):
    src = edge_index[0].astype(jnp.int32)
    dst = edge_index[1].astype(jnp.int32)

    Al1 = _head_reduce_mat(a_l1)
    Ar1 = _head_reduce_mat(a_r1)
    Rep1 = _repeat_mat(H, F1)
    Al2 = _head_reduce_mat(a_l2)
    Ar2 = _head_reduce_mat(a_r2)
    Rep2 = _repeat_mat(H, C)
    S2 = _headsum_mat(H, C)
    b2m = jnp.mean(b2.reshape(H, C), axis=0, keepdims=True)

    h1, el1, er1, w1s, winit1 = _prep1(features, W1, Al1, Ar1, Rep1)
    zeros8 = jnp.zeros((N, 8), _f32)
    htab1 = jnp.concatenate([h1, el1, zeros8], axis=1)
    ert1 = jnp.concatenate([er1, zeros8], axis=1)
    init1 = jnp.stack([
        jnp.concatenate([winit1, w1s, jnp.ones((N, 8), _f32)], axis=1),
        jnp.zeros((N, W1ROW), _f32),
    ])
    init1 = jnp.pad(init1, ((0, 0), (0, NPAD - N), (0, 0)))
    p1 = _edge_kernel(W1ROW, H, E // 2, htab1, ert1, init1, src, dst, dst)
    p1 = p1[:, :N]

    h2, el2, er2, w2s, winit2 = _prep2(p1, b1.reshape(1, HF1), Rep1,
                                       W2, Al2, Ar2, Rep2)
    halves_h, halves_e, halves_i = [], [], []
    zeros12 = jnp.zeros((N, 12), _f32)
    for c in range(NCORE):
        f0, f1b = c * 160, (c + 1) * 160
        h0, h1b = c * 4, (c + 1) * 4
        halves_h.append(jnp.concatenate(
            [h2[:, f0:f1b], el2[:, h0:h1b], zeros12], axis=1))
        halves_e.append(jnp.concatenate(
            [er2[:, h0:h1b], zeros12], axis=1))
        halves_i.append(jnp.concatenate(
            [winit2[:, f0:f1b], w2s[:, h0:h1b], jnp.ones((N, 12), _f32)],
            axis=1))
    htab2 = jnp.concatenate(halves_h, axis=0)
    ert2 = jnp.concatenate(halves_e, axis=0)
    init2 = jnp.pad(jnp.stack(halves_i), ((0, 0), (0, NPAD - N), (0, 0)))
    src2 = jnp.concatenate([src, src + N])
    dst2 = jnp.concatenate([dst, dst + N])
    dst2s = jnp.concatenate([dst, dst])
    p2 = _edge_kernel(W2ROW, 4, E, htab2, ert2, init2, src2, dst2, dst2s)

    nume = jnp.concatenate([p2[0, :N, :160], p2[1, :N, :160]], axis=1)
    dens = jnp.concatenate([p2[0, :N, 160:164], p2[1, :N, 160:164]], axis=1)
    return _final(nume, dens, Rep2, S2, b2m)
